# Initial kernel scaffold; baseline (speedup 1.0000x reference)
import jax
import jax.numpy as jnp
from jax.experimental import pallas as pl


def kernel(head_outputs, anchors):
    raise NotImplementedError("write your pallas kernel here")


# speedup vs baseline: 1.1766x; 1.1766x over previous
import jax
import jax.numpy as jnp
from jax.experimental import pallas as pl
from jax.experimental.pallas import tpu as pltpu

SCORE_THRESH = 0.05
NMS_THRESH = 0.5
DET_PER_IMG = 300
PRE_NMS_TOPK = 1024
N_BOX = 20000
N_CLS = 80
GROUPS = 1250
GROUP_ROWS = 16
GATHER_CHUNK = 2000
BIG = 1 << 30


def _eye(n):
    a = jax.lax.broadcasted_iota(jnp.int32, (n, n), 0)
    b = jax.lax.broadcasted_iota(jnp.int32, (n, n), 1)
    return (a == b).astype(jnp.float32)


def _postprocess_body(pred_ref, boxes_ref, scores_ref, labels_ref,
                      s3_ref, iou_ref):
    pred = pred_ref[0]
    obj = pred[:, 4:5]
    cls = pred[:, 5:5 + N_CLS]
    s = cls * obj
    s = jnp.where(s > SCORE_THRESH, s, 0.0)
    s3 = s.reshape(GROUPS, GROUP_ROWS, N_CLS)
    s3_ref[...] = s3

    gmax_col = jnp.max(jnp.max(s3, axis=1), axis=1, keepdims=True)
    gmaxT = jax.lax.dot_general(gmax_col, _eye(GROUPS),
                                (((0,), (0,)), ((), ())),
                                preferred_element_type=jnp.float32,
            precision=jax.lax.Precision.HIGHEST)
    gmaxT = jnp.concatenate(
        [gmaxT, jnp.full((1, 1280 - GROUPS), -1.0, jnp.float32)], axis=1)

    glane = jax.lax.broadcasted_iota(jnp.int32, (1, 1280), 1)
    sub16 = jax.lax.broadcasted_iota(jnp.int32, (GROUP_ROWS, 1), 0)
    lane80 = jax.lax.broadcasted_iota(jnp.int32, (1, N_CLS), 1)
    lane80b = jax.lax.broadcasted_iota(jnp.int32, (GROUP_ROWS, N_CLS), 1)
    lane1024 = jax.lax.broadcasted_iota(jnp.int32, (1, PRE_NMS_TOPK), 1)

    def ext_body(i, carry):
        gmaxT, sc, idx = carry
        m = jnp.max(gmaxT)
        gsel = jnp.min(jnp.where(gmaxT == m, glane, BIG))
        gsel = jnp.minimum(gsel, GROUPS - 1)
        blk = s3_ref[pl.ds(gsel, 1)].reshape(GROUP_ROWS, N_CLS)
        rblk = jnp.max(blk, axis=1, keepdims=True)
        tsel = jnp.min(jnp.where(rblk == m, sub16, BIG))
        tsel = jnp.minimum(tsel, GROUP_ROWS - 1)
        row = s3_ref[pl.ds(gsel, 1), pl.ds(tsel, 1)].reshape(1, N_CLS)
        csel = jnp.min(jnp.where(row == m, lane80, BIG))
        flat = (gsel * GROUP_ROWS + tsel) * N_CLS + csel
        s3_ref[pl.ds(gsel, 1), pl.ds(tsel, 1)] = jnp.where(
            lane80 == csel, -1.0, row).reshape(1, 1, N_CLS)
        blk_upd = jnp.where((sub16 == tsel) & (lane80b == csel), -1.0, blk)
        gnew = jnp.max(blk_upd)
        gmaxT = jnp.where(glane == gsel, gnew, gmaxT)
        sc = jnp.where(lane1024 == i, m, sc)
        idx = jnp.where(lane1024 == i, flat, idx)
        return gmaxT, sc, idx

    gmaxT, top_scores, top_idx = jax.lax.fori_loop(
        0, PRE_NMS_TOPK, ext_body,
        (gmaxT, jnp.zeros((1, PRE_NMS_TOPK), jnp.float32),
         jnp.zeros((1, PRE_NMS_TOPK), jnp.int32)))

    box_idx = top_idx // N_CLS
    labels = top_idx % N_CLS
    labels_f = labels.astype(jnp.float32)

    p4 = pred[:, 0:4]
    candT = jnp.zeros((4, PRE_NMS_TOPK), jnp.float32)
    for c in range(N_BOX // GATHER_CHUNK):
        row_iota = jax.lax.broadcasted_iota(jnp.int32, (GATHER_CHUNK, 1), 0)
        ohT = (row_iota + c * GATHER_CHUNK == box_idx).astype(jnp.float32)
        chunk = p4[c * GATHER_CHUNK:(c + 1) * GATHER_CHUNK, :]
        candT = candT + jax.lax.dot_general(
            chunk, ohT, (((0,), (0,)), ((), ())),
            preferred_element_type=jnp.float32,
            precision=jax.lax.Precision.HIGHEST)

    cxT, cyT = candT[0:1, :], candT[1:2, :]
    wT, hT = candT[2:3, :], candT[3:4, :]
    x1T = cxT - wT * 0.5
    y1T = cyT - hT * 0.5
    x2T = cxT + wT * 0.5
    y2T = cyT + hT * 0.5
    offT = labels_f * 8.0
    ox1T, oy1T = x1T + offT, y1T + offT
    ox2T, oy2T = x2T + offT, y2T + offT
    areaT = jnp.maximum(x2T - x1T, 0.0) * jnp.maximum(y2T - y1T, 0.0)

    e8 = _eye(8)
    t8 = jnp.concatenate([ox1T, oy1T, ox2T, oy2T, areaT,
                          jnp.zeros((3, PRE_NMS_TOPK), jnp.float32)], axis=0)
    cols = jax.lax.dot_general(t8, e8, (((0,), (0,)), ((), ())),
                               preferred_element_type=jnp.float32,
            precision=jax.lax.Precision.HIGHEST)
    ox1c, oy1c = cols[:, 0:1], cols[:, 1:2]
    ox2c, oy2c = cols[:, 2:3], cols[:, 3:4]
    areac = cols[:, 4:5]

    ix1 = jnp.maximum(ox1c, ox1T)
    iy1 = jnp.maximum(oy1c, oy1T)
    ix2 = jnp.minimum(ox2c, ox2T)
    iy2 = jnp.minimum(oy2c, oy2T)
    inter = jnp.maximum(ix2 - ix1, 0.0) * jnp.maximum(iy2 - iy1, 0.0)
    union = areac + areaT - inter
    iou_ref[...] = inter / jnp.maximum(union, 1e-9)

    valid_f = (top_scores > 0.0).astype(jnp.float32)

    def nms_body(i, keep):
        row = iou_ref[pl.ds(i, 1)]
        hits = keep * jnp.where((row > NMS_THRESH) & (lane1024 < i), 1.0, 0.0)
        sup = jnp.max(hits) > 0.5
        v_i = jnp.sum(jnp.where(lane1024 == i, valid_f, 0.0)) > 0.5
        bit = jnp.where(v_i & jnp.logical_not(sup), 1.0, 0.0)
        return jnp.where(lane1024 == i, bit, keep)

    keep = jax.lax.fori_loop(0, PRE_NMS_TOPK, nms_body,
                             jnp.zeros((1, PRE_NMS_TOPK), jnp.float32))

    rank = jnp.where(keep > 0.5, top_scores, -1.0)
    lane300 = jax.lax.broadcasted_iota(jnp.int32, (1, DET_PER_IMG), 1)

    def sel_body(i, carry):
        rank, sr, si = carry
        m = jnp.max(rank)
        j = jnp.min(jnp.where(rank == m, lane1024, BIG))
        sr = jnp.where(lane300 == i, m, sr)
        si = jnp.where(lane300 == i, j, si)
        rank = jnp.where(lane1024 == j, -2.0, rank)
        return rank, sr, si

    _, sel_rank, sel_idx = jax.lax.fori_loop(
        0, DET_PER_IMG, sel_body,
        (rank, jnp.zeros((1, DET_PER_IMG), jnp.float32),
         jnp.zeros((1, DET_PER_IMG), jnp.int32)))

    payT = jnp.concatenate([x1T, y1T, x2T, y2T, labels_f, top_scores,
                            jnp.zeros((2, PRE_NMS_TOPK), jnp.float32)], axis=0)
    cand_iota = jax.lax.broadcasted_iota(jnp.int32, (PRE_NMS_TOPK, 1), 0)
    ohT2 = (cand_iota == sel_idx).astype(jnp.float32)
    selT = jax.lax.dot_general(payT, ohT2, (((1,), (0,)), ((), ())),
                               preferred_element_type=jnp.float32,
            precision=jax.lax.Precision.HIGHEST)
    sel8 = jax.lax.dot_general(selT, e8, (((0,), (0,)), ((), ())),
                               preferred_element_type=jnp.float32,
            precision=jax.lax.Precision.HIGHEST)

    mask_lane = sel_rank > 0.0
    mask_col = sel8[:, 5:6] > 0.0

    det_boxes = jnp.where(mask_col, sel8[:, 0:4], 0.0)
    det_scores = jnp.where(mask_lane, sel_rank, 0.0)
    lab_sel = (selT[4:5, :] + 0.5).astype(jnp.int32)
    det_labels = jnp.where(mask_lane, lab_sel, -1)

    boxes_ref[...] = det_boxes.reshape(1, DET_PER_IMG, 4)
    scores_ref[...] = det_scores.reshape(1, 1, DET_PER_IMG)
    labels_ref[...] = det_labels.reshape(1, 1, DET_PER_IMG)


@jax.jit
def kernel(head_outputs, anchors):
    del anchors
    n = head_outputs.shape[0]
    boxes, scores, labels = pl.pallas_call(
        _postprocess_body,
        grid=(n,),
        in_specs=[pl.BlockSpec((1, N_BOX, 85), lambda i: (i, 0, 0))],
        out_specs=[
            pl.BlockSpec((1, DET_PER_IMG, 4), lambda i: (i, 0, 0)),
            pl.BlockSpec((1, 1, DET_PER_IMG), lambda i: (i, 0, 0)),
            pl.BlockSpec((1, 1, DET_PER_IMG), lambda i: (i, 0, 0)),
        ],
        out_shape=[
            jax.ShapeDtypeStruct((n, DET_PER_IMG, 4), jnp.float32),
            jax.ShapeDtypeStruct((n, 1, DET_PER_IMG), jnp.float32),
            jax.ShapeDtypeStruct((n, 1, DET_PER_IMG), jnp.int32),
        ],
        scratch_shapes=[
            pltpu.VMEM((GROUPS, GROUP_ROWS, N_CLS), jnp.float32),
            pltpu.VMEM((PRE_NMS_TOPK, PRE_NMS_TOPK), jnp.float32),
        ],
    )(head_outputs)
    return boxes, scores.reshape(n, DET_PER_IMG), labels.reshape(n, DET_PER_IMG)
